# Initial kernel scaffold; baseline (speedup 1.0000x reference)
import jax
import jax.numpy as jnp
from jax.experimental import pallas as pl


def kernel(x, xyz, idx, att_w, momentum_w, momentum_bn_g, momentum_bn_b, agent_w, agent_bn_g, agent_bn_b):
    raise NotImplementedError("write your pallas kernel here")


# speedup vs baseline: 2.9925x; 2.9925x over previous
import functools

import jax
import jax.numpy as jnp
import numpy as np
from jax import lax
from jax.experimental import pallas as pl
from jax.experimental.pallas import tpu as pltpu
from jax.experimental.pallas import tpu_sc as plsc

BS, N, C, K, CN, CL = 8, 4096, 128, 32, 256, 5
BN_EPS = 1e-5
M = BS * CN
CPW = CN // 4
NC8 = C // 16

_MW0X, _MW0P, _MW1X, _MW1P = 0, 128, 256, 384
_AWX, _AWP = 512, 640
_CONST = 768
_PAR_LEN = 776
_PAR_PAD = 784


def _sqrtv(x):
    bits = lax.bitcast_convert_type(x, jnp.int32)
    y0b = jnp.int32(0x1FBD1DF5) + lax.shift_right_logical(bits, 1)
    y = lax.bitcast_convert_type(y0b, jnp.float32)
    for _ in range(3):
        y = 0.5 * (y + x / y)
    return y


def _bf16r(x):
    b = lax.bitcast_convert_type(x, jnp.int32)
    r = b + jnp.int32(0x7FFF) + (lax.shift_right_logical(b, 16) & 1)
    return lax.bitcast_convert_type(r & jnp.int32(-65536), jnp.float32)


def _splat(ref, i):
    return plsc.load_gather(ref, [jnp.broadcast_to(i, (16,)).astype(jnp.int32)])


def _splat2(ref, i, k):
    ii = jnp.broadcast_to(i, (16,)).astype(jnp.int32)
    kk = jnp.broadcast_to(k, (16,)).astype(jnp.int32)
    return plsc.load_gather(ref, [ii, kk])


def _argmax32(l0, l1, iota):
    mg = jnp.max(jnp.maximum(l0, l1))
    big = jnp.full((16,), 99, jnp.int32)
    i0 = jnp.min(jnp.where(l0 == mg, iota, big))
    i1 = jnp.min(jnp.where(l1 == mg, iota + 16, big))
    return jnp.minimum(i0, i1)


def _walk_body(xw_hbm, adj_hbm, start_hbm, par_hbm, out_hbm,
               par_v, q_v, start_v, cur_idx_v, idx4_v, nk_v, nkw_v, nkf_v,
               cur_rows, pre_rows, dir_rows, nbr, mm0_v, mm1_v,
               mma_v, mmb_v, w0_v, w1_v, n1_v, qp_v, dn_v,
               q_sh, mm_sh, sem):
    cc = lax.axis_index("c")
    ss = lax.axis_index("s")
    b = cc * 4 + ss // 4
    j = ss % 4
    bslot = ss // 4
    wslot = b * 4 + j
    iota = lax.broadcasted_iota(jnp.int32, (16,), 0)
    lane0 = iota == 0
    zero = jnp.zeros((16,), jnp.float32)
    bN = jnp.broadcast_to(b * N, (16,)).astype(jnp.int32)

    def put(ref, i, val):
        v = jnp.broadcast_to(val, (16,)).astype(ref.dtype)
        ii = jnp.broadcast_to(i, (16,)).astype(jnp.int32)
        plsc.store_scatter(ref, [ii], v, mask=lane0)

    pltpu.sync_copy(par_hbm, par_v)
    mw0x = [par_v[pl.ds(_MW0X + 16 * c, 16)] for c in range(NC8)]
    mw0p = [par_v[pl.ds(_MW0P + 16 * c, 16)] for c in range(NC8)]
    mw1x = [par_v[pl.ds(_MW1X + 16 * c, 16)] for c in range(NC8)]
    mw1p = [par_v[pl.ds(_MW1P + 16 * c, 16)] for c in range(NC8)]
    awx = [par_v[pl.ds(_AWX + 16 * c, 16)] for c in range(NC8)]
    awp = [par_v[pl.ds(_AWP + 16 * c, 16)] for c in range(NC8)]
    cg_m0 = _splat(par_v, _CONST + 0)
    b_m0 = _splat(par_v, _CONST + 1)
    cg_m1 = _splat(par_v, _CONST + 2)
    b_m1 = _splat(par_v, _CONST + 3)
    cg_a = _splat(par_v, _CONST + 4)
    b_a = _splat(par_v, _CONST + 5)

    def q_of(i, h):
        return plsc.load_gather(q_v, [nk_v[i, pl.ds(16 * h, 16)]])

    def fetch_neighbors():
        for g in range(4):
            idx4_v[pl.ds(16 * g, 16)] = cur_idx_v[pl.ds(16 * g, 16)] // 4
        pltpu.async_copy(adj_hbm.at[idx4_v], nkw_v, sem).wait()

        def nk_curve(i, _):
            off = ((_splat(cur_idx_v, i) % 4) * 32)[0]
            nk_v[i, pl.ds(0, 16)] = nkw_v[i, pl.ds(off, 16)]
            nk_v[i, pl.ds(16, 16)] = nkw_v[i, pl.ds(off + 16, 16)]
            return 0

        lax.fori_loop(0, CPW, nk_curve, 0)

    qrow0 = b * N + j * 1024

    def q_chunk(ch, _):
        pltpu.async_copy(xw_hbm.at[pl.ds(qrow0 + ch * 128, 128)], nbr,
                         sem).wait()

        def q_row(r, _):
            acc = zero
            for c in range(NC8):
                acc = acc + _bf16r(nbr[r, pl.ds(16 * c, 16)]) * awx[c]
            put(q_v, j * 1024 + ch * 128 + r, jnp.sum(acc))
            return 0

        lax.fori_loop(0, 128, q_row, 0)
        return 0

    lax.fori_loop(0, 8, q_chunk, 0)
    pltpu.sync_copy(q_v.at[pl.ds(j * 1024, 1024)],
                    q_sh.at[bslot, pl.ds(j * 1024, 1024)])
    plsc.subcore_barrier()
    pltpu.sync_copy(q_sh.at[bslot], q_v)
    plsc.subcore_barrier()

    pltpu.sync_copy(start_hbm.at[wslot], start_v)
    for g in range(4):
        cur_idx_v[pl.ds(16 * g, 16)] = start_v[pl.ds(16 * g, 16)] + bN
    pltpu.async_copy(xw_hbm.at[cur_idx_v], pre_rows, sem).wait()
    fetch_neighbors()

    def s0_curve(i, _):
        acc = zero
        for c in range(NC8):
            acc = acc + _bf16r(pre_rows[i, pl.ds(16 * c, 16)]) * awp[c]
        qpre = jnp.broadcast_to(jnp.sum(acc), (16,))
        l0 = (q_of(i, 0) + qpre) * cg_a + b_a
        l1 = (q_of(i, 1) + qpre) * cg_a + b_a
        kstar = _argmax32(l0, l1, iota)
        put(cur_idx_v, i, _splat2(nk_v, i, kstar) + bN)
        return 0

    lax.fori_loop(0, CPW, s0_curve, 0)
    pltpu.async_copy(xw_hbm.at[cur_idx_v], cur_rows, sem).wait()
    out0 = b * CN + j * CPW
    pltpu.sync_copy(cur_rows, out_hbm.at[pl.ds(out0, CPW)])

    def step(t, _):
        def mm_curve(i, _):
            a0 = zero
            a1 = zero
            for c in range(NC8):
                cu = _bf16r(cur_rows[i, pl.ds(16 * c, 16)])
                pr = _bf16r(pre_rows[i, pl.ds(16 * c, 16)])
                a0 = a0 + cu * mw0x[c] + pr * mw0p[c]
                a1 = a1 + cu * mw1x[c] + pr * mw1p[c]
            s0 = jnp.broadcast_to(jnp.sum(a0), (16,))
            s1 = jnp.broadcast_to(jnp.sum(a1), (16,))
            put(mma_v, i, s0 * cg_m0 + b_m0)
            put(mmb_v, i, s1 * cg_m1 + b_m1)
            return 0

        lax.fori_loop(0, CPW, mm_curve, 0)
        pltpu.sync_copy(mma_v, mm_sh.at[bslot, 0, pl.ds(j * CPW, CPW)])
        pltpu.sync_copy(mmb_v, mm_sh.at[bslot, 1, pl.ds(j * CPW, CPW)])
        plsc.subcore_barrier()
        half = (j % 2) * 128
        pltpu.sync_copy(mm_sh.at[bslot, 0, pl.ds(half, 128)], mm0_v)
        pltpu.sync_copy(mm_sh.at[bslot, 1, pl.ds(half, 128)], mm1_v)
        plsc.subcore_barrier()
        hi = jnp.broadcast_to(j >= 2, (16,))
        for g in range(4):
            ev = 2 * (iota + 16 * g)
            od = ev + 1
            a0e = plsc.load_gather(mm0_v, [ev])
            a1e = plsc.load_gather(mm1_v, [ev])
            a0o = plsc.load_gather(mm0_v, [od])
            a1o = plsc.load_gather(mm1_v, [od])
            mxe = jnp.maximum(a0e, a1e)
            e0e = jnp.exp(a0e - mxe)
            e1e = jnp.exp(a1e - mxe)
            mxo = jnp.maximum(a0o, a1o)
            e0o = jnp.exp(a0o - mxo)
            e1o = jnp.exp(a1o - mxo)
            sel_e = jnp.where(hi, e1e, e0e)
            sel_o = jnp.where(hi, e1o, e0o)
            w0_v[pl.ds(16 * g, 16)] = sel_e / (e0e + e1e)
            w1_v[pl.ds(16 * g, 16)] = sel_o / (e0o + e1o)

        def pre_curve(i, _):
            w0 = _splat(w0_v, i)
            w1 = _splat(w1_v, i)
            an = zero
            aq = zero
            for c in range(NC8):
                cu = cur_rows[i, pl.ds(16 * c, 16)]
                pr = pre_rows[i, pl.ds(16 * c, 16)]
                pn = w0 * cu + w1 * pr
                dv = cu - pn
                pre_rows[i, pl.ds(16 * c, 16)] = pn
                dir_rows[i, pl.ds(16 * c, 16)] = dv
                an = an + dv * dv
                aq = aq + _bf16r(pn) * awp[c]
            put(n1_v, i, jnp.sum(an))
            put(qp_v, i, jnp.sum(aq))
            return 0

        lax.fori_loop(0, CPW, pre_curve, 0)
        for g in range(4):
            n1_v[pl.ds(16 * g, 16)] = _sqrtv(n1_v[pl.ds(16 * g, 16)])

        fetch_neighbors()

        def nkf_curve(i, _):
            r = i // 4
            o = 32 * (i % 4)
            nkf_v[r, pl.ds(o, 16)] = nk_v[i, pl.ds(0, 16)] + bN
            nkf_v[r, pl.ds(o + 16, 16)] = nk_v[i, pl.ds(16, 16)] + bN
            return 0

        lax.fori_loop(0, CPW, nkf_curve, 0)

        def chunk(ch, _):
            pltpu.async_copy(xw_hbm.at[nkf_v.at[ch]], nbr, sem).wait()

            def curve(ci, _):
                i = 4 * ch + ci
                curv = [cur_rows[i, pl.ds(16 * c, 16)] for c in range(NC8)]
                dirb = [_bf16r(dir_rows[i, pl.ds(16 * c, 16)])
                        for c in range(NC8)]

                def krow(kg, _):
                    for ku in range(4):
                        kk = 4 * kg + ku
                        ad = zero
                        an = zero
                        for c in range(NC8):
                            e = nbr[32 * ci + kk, pl.ds(16 * c, 16)] - curv[c]
                            ad = ad + _bf16r(e) * dirb[c]
                            an = an + e * e
                        put(dn_v, kk, jnp.sum(ad))
                        put(dn_v, 32 + kk, jnp.sum(an))
                    return 0

                lax.fori_loop(0, 8, krow, 0)
                qpre = _splat(qp_v, i)
                n1 = _splat(n1_v, i)
                ls = []
                for h in range(2):
                    dot = dn_v[pl.ds(16 * h, 16)]
                    n2 = _sqrtv(dn_v[pl.ds(32 + 16 * h, 16)])
                    den = jnp.maximum(n1 * n2, 1e-8)
                    d = jnp.clip(1.0 + dot / den, 0.0, 1.0)
                    ls.append(((q_of(i, h) + qpre) * cg_a + b_a) * d)
                kstar = _argmax32(ls[0], ls[1], iota)
                put(cur_idx_v, i, _splat2(nk_v, i, kstar) + bN)
                row = 32 * ci + kstar
                for c in range(NC8):
                    cur_rows[i, pl.ds(16 * c, 16)] = nbr[row, pl.ds(16 * c, 16)]
                return 0

            lax.fori_loop(0, 4, curve, 0)
            return 0

        lax.fori_loop(0, 16, chunk, 0)
        o = t * M + b * CN + j * CPW
        pltpu.sync_copy(cur_rows, out_hbm.at[pl.ds(o, CPW)])
        return 0

    lax.fori_loop(1, CL, step, 0)


@jax.jit
def _run(xw_t, adj, start, par):
    mesh = plsc.VectorSubcoreMesh(core_axis_name="c", subcore_axis_name="s")
    walk = functools.partial(
        pl.kernel,
        mesh=mesh,
        out_type=jax.ShapeDtypeStruct((CL * M, C), jnp.float32),
        compiler_params=pltpu.CompilerParams(needs_layout_passes=False),
        scratch_types=[
            pltpu.VMEM((_PAR_PAD,), jnp.float32),
            pltpu.VMEM((N,), jnp.float32),
            pltpu.VMEM((CPW,), jnp.int32),
            pltpu.VMEM((CPW,), jnp.int32),
            pltpu.VMEM((CPW,), jnp.int32),
            pltpu.VMEM((CPW, K), jnp.int32),
            pltpu.VMEM((CPW, 128), jnp.int32),
            pltpu.VMEM((16, 128), jnp.int32),
            pltpu.VMEM((CPW, C), jnp.float32),
            pltpu.VMEM((CPW, C), jnp.float32),
            pltpu.VMEM((CPW, C), jnp.float32),
            pltpu.VMEM((128, C), jnp.float32),
            pltpu.VMEM((128,), jnp.float32),
            pltpu.VMEM((128,), jnp.float32),
            pltpu.VMEM((CPW,), jnp.float32),
            pltpu.VMEM((CPW,), jnp.float32),
            pltpu.VMEM((CPW,), jnp.float32),
            pltpu.VMEM((CPW,), jnp.float32),
            pltpu.VMEM((CPW,), jnp.float32),
            pltpu.VMEM((CPW,), jnp.float32),
            pltpu.VMEM((64,), jnp.float32),
            pltpu.VMEM_SHARED((4, N), jnp.float32),
            pltpu.VMEM_SHARED((4, 2, CN), jnp.float32),
            pltpu.SemaphoreType.DMA,
        ],
    )(_walk_body)
    return walk(xw_t, adj, start, par)


def kernel(x, xyz, idx, att_w, momentum_w, momentum_bn_g, momentum_bn_b,
           agent_w, agent_bn_g, agent_bn_b):
    x_att = jax.nn.sigmoid(jnp.einsum('bcn,oc->bon', x, att_w[:, :, 0]))
    _, start_index = jax.lax.top_k(x_att, CN)
    start = start_index[:, 0, :].reshape(BS * 4, CN // 4).astype(jnp.int32)

    xw_t = (x * x_att).transpose(0, 2, 1).reshape(BS * N, C)
    adj = idx.reshape(BS * N // 4, 4 * K)

    bns = np.float32(1.0 / np.sqrt(1.0 + BN_EPS))
    mw = momentum_w[:, :, 0]
    aw = agent_w[0, :, 0, 0]
    consts = jnp.stack([
        bns * momentum_bn_g[0], momentum_bn_b[0],
        bns * momentum_bn_g[1], momentum_bn_b[1],
        bns * agent_bn_g[0], agent_bn_b[0],
        jnp.float32(0.0), jnp.float32(0.0)])
    wblk = jnp.concatenate([mw[0, :C], mw[0, C:], mw[1, :C], mw[1, C:],
                            aw[:C], aw[C:]])
    wblk = wblk.astype(jnp.bfloat16).astype(jnp.float32)
    par = jnp.concatenate([
        wblk, consts,
        jnp.zeros((_PAR_PAD - _PAR_LEN,), jnp.float32)]).astype(jnp.float32)

    out_rows = _run(xw_t, adj, start, par)
    return out_rows.reshape(CL, BS, CN, C).transpose(1, 3, 2, 0)


# speedup vs baseline: 3.5129x; 1.1739x over previous
import functools

import jax
import jax.numpy as jnp
import numpy as np
from jax import lax
from jax.experimental import pallas as pl
from jax.experimental.pallas import tpu as pltpu
from jax.experimental.pallas import tpu_sc as plsc

BS, N, C, K, CN, CL = 8, 4096, 128, 32, 256, 5
BN_EPS = 1e-5
M = BS * CN
CPW = CN // 4
NC8 = C // 16

_MW0X, _MW0P, _MW1X, _MW1P = 0, 128, 256, 384
_AWX, _AWP = 512, 640
_CONST = 768
_PAR_LEN = 776
_PAR_PAD = 784


def _sqrtv(x):
    bits = lax.bitcast_convert_type(x, jnp.int32)
    y0b = jnp.int32(0x1FBD1DF5) + lax.shift_right_logical(bits, 1)
    y = lax.bitcast_convert_type(y0b, jnp.float32)
    for _ in range(3):
        y = 0.5 * (y + x / y)
    return y


def _bf16r(x):
    b = lax.bitcast_convert_type(x, jnp.int32)
    r = b + jnp.int32(0x7FFF) + (lax.shift_right_logical(b, 16) & 1)
    return lax.bitcast_convert_type(r & jnp.int32(-65536), jnp.float32)


def _splat(ref, i):
    return plsc.load_gather(ref, [jnp.broadcast_to(i, (16,)).astype(jnp.int32)])


def _splat2(ref, i, k):
    ii = jnp.broadcast_to(i, (16,)).astype(jnp.int32)
    kk = jnp.broadcast_to(k, (16,)).astype(jnp.int32)
    return plsc.load_gather(ref, [ii, kk])


def _argmax32(l0, l1, iota):
    mg = jnp.max(jnp.maximum(l0, l1))
    big = jnp.full((16,), 99, jnp.int32)
    i0 = jnp.min(jnp.where(l0 == mg, iota, big))
    i1 = jnp.min(jnp.where(l1 == mg, iota + 16, big))
    return jnp.minimum(i0, i1)


def _walk_body(xw_hbm, adj_hbm, start_hbm, par_hbm, out_hbm,
               par_v, q_v, start_v, cur_idx_v, idx4_v, nk_v, nkw_v, nkf_v,
               cur_rows, pre_rows, dir_rows, nbr, nbr2, mm0_v, mm1_v,
               mma_v, mmb_v, w0_v, w1_v, n1_v, qp_v, dn_v,
               q_sh, mm_sh, sem, sem2):
    cc = lax.axis_index("c")
    ss = lax.axis_index("s")
    b = cc * 4 + ss // 4
    j = ss % 4
    bslot = ss // 4
    wslot = b * 4 + j
    iota = lax.broadcasted_iota(jnp.int32, (16,), 0)
    lane0 = iota == 0
    zero = jnp.zeros((16,), jnp.float32)
    bN = jnp.broadcast_to(b * N, (16,)).astype(jnp.int32)

    def put(ref, i, val):
        v = jnp.broadcast_to(val, (16,)).astype(ref.dtype)
        ii = jnp.broadcast_to(i, (16,)).astype(jnp.int32)
        plsc.store_scatter(ref, [ii], v, mask=lane0)

    pltpu.sync_copy(par_hbm, par_v)
    mw0x = [par_v[pl.ds(_MW0X + 16 * c, 16)] for c in range(NC8)]
    mw0p = [par_v[pl.ds(_MW0P + 16 * c, 16)] for c in range(NC8)]
    mw1x = [par_v[pl.ds(_MW1X + 16 * c, 16)] for c in range(NC8)]
    mw1p = [par_v[pl.ds(_MW1P + 16 * c, 16)] for c in range(NC8)]
    awx = [par_v[pl.ds(_AWX + 16 * c, 16)] for c in range(NC8)]
    awp = [par_v[pl.ds(_AWP + 16 * c, 16)] for c in range(NC8)]
    cg_m0 = _splat(par_v, _CONST + 0)
    b_m0 = _splat(par_v, _CONST + 1)
    cg_m1 = _splat(par_v, _CONST + 2)
    b_m1 = _splat(par_v, _CONST + 3)
    cg_a = _splat(par_v, _CONST + 4)
    b_a = _splat(par_v, _CONST + 5)

    def q_of(i, h):
        return plsc.load_gather(q_v, [nk_v[i, pl.ds(16 * h, 16)]])

    def fetch_neighbors():
        for g in range(4):
            idx4_v[pl.ds(16 * g, 16)] = cur_idx_v[pl.ds(16 * g, 16)] // 4
        pltpu.async_copy(adj_hbm.at[idx4_v], nkw_v, sem).wait()

        def nk_curve(i, _):
            off = ((_splat(cur_idx_v, i) % 4) * 32)[0]
            nk_v[i, pl.ds(0, 16)] = nkw_v[i, pl.ds(off, 16)]
            nk_v[i, pl.ds(16, 16)] = nkw_v[i, pl.ds(off + 16, 16)]
            return 0

        lax.fori_loop(0, CPW, nk_curve, 0)

    qrow0 = b * N + j * 1024

    def q_chunk(ch, _):
        pltpu.async_copy(xw_hbm.at[pl.ds(qrow0 + ch * 128, 128)], nbr,
                         sem).wait()

        def q_row(r, _):
            acc = zero
            for c in range(NC8):
                acc = acc + _bf16r(nbr[r, pl.ds(16 * c, 16)]) * awx[c]
            put(q_v, j * 1024 + ch * 128 + r, jnp.sum(acc))
            return 0

        lax.fori_loop(0, 128, q_row, 0)
        return 0

    lax.fori_loop(0, 8, q_chunk, 0)
    pltpu.sync_copy(q_v.at[pl.ds(j * 1024, 1024)],
                    q_sh.at[bslot, pl.ds(j * 1024, 1024)])
    plsc.subcore_barrier()
    pltpu.sync_copy(q_sh.at[bslot], q_v)
    plsc.subcore_barrier()

    pltpu.sync_copy(start_hbm.at[wslot], start_v)
    for g in range(4):
        cur_idx_v[pl.ds(16 * g, 16)] = start_v[pl.ds(16 * g, 16)] + bN
    pltpu.async_copy(xw_hbm.at[cur_idx_v], pre_rows, sem).wait()
    fetch_neighbors()

    def s0_curve(i, _):
        acc = zero
        for c in range(NC8):
            acc = acc + _bf16r(pre_rows[i, pl.ds(16 * c, 16)]) * awp[c]
        qpre = jnp.broadcast_to(jnp.sum(acc), (16,))
        l0 = (q_of(i, 0) + qpre) * cg_a + b_a
        l1 = (q_of(i, 1) + qpre) * cg_a + b_a
        kstar = _argmax32(l0, l1, iota)
        put(cur_idx_v, i, _splat2(nk_v, i, kstar) + bN)
        return 0

    lax.fori_loop(0, CPW, s0_curve, 0)
    pltpu.async_copy(xw_hbm.at[cur_idx_v], cur_rows, sem).wait()
    out0 = b * CN + j * CPW
    pltpu.sync_copy(cur_rows, out_hbm.at[pl.ds(out0, CPW)])

    def step(t, _):
        def mm_curve(i, _):
            a0 = zero
            a1 = zero
            for c in range(NC8):
                cu = _bf16r(cur_rows[i, pl.ds(16 * c, 16)])
                pr = _bf16r(pre_rows[i, pl.ds(16 * c, 16)])
                a0 = a0 + cu * mw0x[c] + pr * mw0p[c]
                a1 = a1 + cu * mw1x[c] + pr * mw1p[c]
            s0 = jnp.broadcast_to(jnp.sum(a0), (16,))
            s1 = jnp.broadcast_to(jnp.sum(a1), (16,))
            put(mma_v, i, s0 * cg_m0 + b_m0)
            put(mmb_v, i, s1 * cg_m1 + b_m1)
            return 0

        lax.fori_loop(0, CPW, mm_curve, 0)
        pltpu.sync_copy(mma_v, mm_sh.at[bslot, 0, pl.ds(j * CPW, CPW)])
        pltpu.sync_copy(mmb_v, mm_sh.at[bslot, 1, pl.ds(j * CPW, CPW)])
        plsc.subcore_barrier()
        half = (j % 2) * 128
        pltpu.sync_copy(mm_sh.at[bslot, 0, pl.ds(half, 128)], mm0_v)
        pltpu.sync_copy(mm_sh.at[bslot, 1, pl.ds(half, 128)], mm1_v)
        plsc.subcore_barrier()
        hi = jnp.broadcast_to(j >= 2, (16,))
        for g in range(4):
            ev = 2 * (iota + 16 * g)
            od = ev + 1
            a0e = plsc.load_gather(mm0_v, [ev])
            a1e = plsc.load_gather(mm1_v, [ev])
            a0o = plsc.load_gather(mm0_v, [od])
            a1o = plsc.load_gather(mm1_v, [od])
            mxe = jnp.maximum(a0e, a1e)
            e0e = jnp.exp(a0e - mxe)
            e1e = jnp.exp(a1e - mxe)
            mxo = jnp.maximum(a0o, a1o)
            e0o = jnp.exp(a0o - mxo)
            e1o = jnp.exp(a1o - mxo)
            sel_e = jnp.where(hi, e1e, e0e)
            sel_o = jnp.where(hi, e1o, e0o)
            w0_v[pl.ds(16 * g, 16)] = sel_e / (e0e + e1e)
            w1_v[pl.ds(16 * g, 16)] = sel_o / (e0o + e1o)

        def pre_curve(i, _):
            w0 = _splat(w0_v, i)
            w1 = _splat(w1_v, i)
            an = zero
            aq = zero
            for c in range(NC8):
                cu = cur_rows[i, pl.ds(16 * c, 16)]
                pr = pre_rows[i, pl.ds(16 * c, 16)]
                pn = w0 * cu + w1 * pr
                dv = cu - pn
                pre_rows[i, pl.ds(16 * c, 16)] = pn
                dir_rows[i, pl.ds(16 * c, 16)] = dv
                an = an + dv * dv
                aq = aq + _bf16r(pn) * awp[c]
            put(n1_v, i, jnp.sum(an))
            put(qp_v, i, jnp.sum(aq))
            return 0

        lax.fori_loop(0, CPW, pre_curve, 0)
        for g in range(4):
            n1_v[pl.ds(16 * g, 16)] = _sqrtv(n1_v[pl.ds(16 * g, 16)])

        fetch_neighbors()

        def nkf_curve(i, _):
            r = i // 4
            o = 32 * (i % 4)
            nkf_v[r, pl.ds(o, 16)] = nk_v[i, pl.ds(0, 16)] + bN
            nkf_v[r, pl.ds(o + 16, 16)] = nk_v[i, pl.ds(16, 16)] + bN
            return 0

        lax.fori_loop(0, CPW, nkf_curve, 0)

        def process(ch, buf):
            def curve(ci, _):
                i = 4 * ch + ci
                curv = [cur_rows[i, pl.ds(16 * c, 16)] for c in range(NC8)]
                dirb = [_bf16r(dir_rows[i, pl.ds(16 * c, 16)])
                        for c in range(NC8)]

                def krow(kg, _):
                    for ku in range(4):
                        kk = 4 * kg + ku
                        ad = zero
                        an = zero
                        for c in range(NC8):
                            e = buf[32 * ci + kk, pl.ds(16 * c, 16)] - curv[c]
                            ad = ad + _bf16r(e) * dirb[c]
                            an = an + e * e
                        put(dn_v, kk, jnp.sum(ad))
                        put(dn_v, 32 + kk, jnp.sum(an))
                    return 0

                lax.fori_loop(0, 8, krow, 0)
                qpre = _splat(qp_v, i)
                n1 = _splat(n1_v, i)
                ls = []
                for h in range(2):
                    dot = dn_v[pl.ds(16 * h, 16)]
                    n2 = _sqrtv(dn_v[pl.ds(32 + 16 * h, 16)])
                    den = jnp.maximum(n1 * n2, 1e-8)
                    d = jnp.clip(1.0 + dot / den, 0.0, 1.0)
                    ls.append(((q_of(i, h) + qpre) * cg_a + b_a) * d)
                kstar = _argmax32(ls[0], ls[1], iota)
                put(cur_idx_v, i, _splat2(nk_v, i, kstar) + bN)
                row = 32 * ci + kstar
                for c in range(NC8):
                    cur_rows[i, pl.ds(16 * c, 16)] = buf[row, pl.ds(16 * c, 16)]
                return 0

            lax.fori_loop(0, 4, curve, 0)

        def fire(ch, buf, fsem):
            pltpu.async_copy(xw_hbm.at[nkf_v.at[ch]], buf, fsem)

        def drain(buf, fsem):
            pltpu.make_async_copy(xw_hbm.at[pl.ds(0, 128)], buf, fsem).wait()

        fire(0, nbr, sem)

        def pair(g, _):
            fire(2 * g + 1, nbr2, sem2)
            drain(nbr, sem)
            process(2 * g, nbr)

            @pl.when(g < 7)
            def _():
                fire(2 * g + 2, nbr, sem)

            drain(nbr2, sem2)
            process(2 * g + 1, nbr2)
            return 0

        lax.fori_loop(0, 8, pair, 0)
        o = t * M + b * CN + j * CPW
        pltpu.sync_copy(cur_rows, out_hbm.at[pl.ds(o, CPW)])
        return 0

    lax.fori_loop(1, CL, step, 0)


@jax.jit
def _run(xw_t, adj, start, par):
    mesh = plsc.VectorSubcoreMesh(core_axis_name="c", subcore_axis_name="s")
    walk = functools.partial(
        pl.kernel,
        mesh=mesh,
        out_type=jax.ShapeDtypeStruct((CL * M, C), jnp.float32),
        compiler_params=pltpu.CompilerParams(needs_layout_passes=False),
        scratch_types=[
            pltpu.VMEM((_PAR_PAD,), jnp.float32),
            pltpu.VMEM((N,), jnp.float32),
            pltpu.VMEM((CPW,), jnp.int32),
            pltpu.VMEM((CPW,), jnp.int32),
            pltpu.VMEM((CPW,), jnp.int32),
            pltpu.VMEM((CPW, K), jnp.int32),
            pltpu.VMEM((CPW, 128), jnp.int32),
            pltpu.VMEM((16, 128), jnp.int32),
            pltpu.VMEM((CPW, C), jnp.float32),
            pltpu.VMEM((CPW, C), jnp.float32),
            pltpu.VMEM((CPW, C), jnp.float32),
            pltpu.VMEM((128, C), jnp.float32),
            pltpu.VMEM((128, C), jnp.float32),
            pltpu.VMEM((128,), jnp.float32),
            pltpu.VMEM((128,), jnp.float32),
            pltpu.VMEM((CPW,), jnp.float32),
            pltpu.VMEM((CPW,), jnp.float32),
            pltpu.VMEM((CPW,), jnp.float32),
            pltpu.VMEM((CPW,), jnp.float32),
            pltpu.VMEM((CPW,), jnp.float32),
            pltpu.VMEM((CPW,), jnp.float32),
            pltpu.VMEM((64,), jnp.float32),
            pltpu.VMEM_SHARED((4, N), jnp.float32),
            pltpu.VMEM_SHARED((4, 2, CN), jnp.float32),
            pltpu.SemaphoreType.DMA,
            pltpu.SemaphoreType.DMA,
        ],
    )(_walk_body)
    return walk(xw_t, adj, start, par)


def kernel(x, xyz, idx, att_w, momentum_w, momentum_bn_g, momentum_bn_b,
           agent_w, agent_bn_g, agent_bn_b):
    x_att = jax.nn.sigmoid(jnp.einsum('bcn,oc->bon', x, att_w[:, :, 0]))
    _, start_index = jax.lax.top_k(x_att, CN)
    start = start_index[:, 0, :].reshape(BS * 4, CN // 4).astype(jnp.int32)

    xw_t = (x * x_att).transpose(0, 2, 1).reshape(BS * N, C)
    adj = idx.reshape(BS * N // 4, 4 * K)

    bns = np.float32(1.0 / np.sqrt(1.0 + BN_EPS))
    mw = momentum_w[:, :, 0]
    aw = agent_w[0, :, 0, 0]
    consts = jnp.stack([
        bns * momentum_bn_g[0], momentum_bn_b[0],
        bns * momentum_bn_g[1], momentum_bn_b[1],
        bns * agent_bn_g[0], agent_bn_b[0],
        jnp.float32(0.0), jnp.float32(0.0)])
    wblk = jnp.concatenate([mw[0, :C], mw[0, C:], mw[1, :C], mw[1, C:],
                            aw[:C], aw[C:]])
    wblk = wblk.astype(jnp.bfloat16).astype(jnp.float32)
    par = jnp.concatenate([
        wblk, consts,
        jnp.zeros((_PAR_PAD - _PAR_LEN,), jnp.float32)]).astype(jnp.float32)

    out_rows = _run(xw_t, adj, start, par)
    return out_rows.reshape(CL, BS, CN, C).transpose(1, 3, 2, 0)
